# baseline (speedup 1.0000x reference)
import functools

import jax
import jax.numpy as jnp
from jax import lax
from jax.experimental import pallas as pl
from jax.experimental.pallas import tpu as pltpu
from jax.experimental.pallas import tpu_sc as plsc

_L = 16
_BLK = 128
_PARAMS = pltpu.CompilerParams(
    use_tc_tiling_on_sc=True, needs_layout_passes=False
)


def _build_pack(D, V, NC, NW):
    TV = (V + _BLK - 1) // _BLK
    VP = TV * _BLK
    mesh = plsc.VectorSubcoreMesh(core_axis_name="c", subcore_axis_name="s")

    @functools.partial(
        pl.kernel,
        out_type=jax.ShapeDtypeStruct((VP // 2, 2 * D), jnp.float32),
        mesh=mesh,
        scratch_types=[
            pltpu.VMEM((2, D, _BLK), jnp.float32),
            pltpu.VMEM((2, _BLK // 2, 2 * D), jnp.float32),
            pltpu.SemaphoreType.DMA((2,)),
            pltpu.SemaphoreType.DMA((2,)),
        ],
        compiler_params=_PARAMS,
    )
    def pack_kernel(tab_hbm, scr_hbm, in_v, out_v, isem, osem):
        wid = lax.axis_index("s") * NC + lax.axis_index("c")
        iota = lax.iota(jnp.int32, _L)
        dvecs = [g * _L + iota for g in range(D // _L)]
        rem = V - (TV - 1) * _BLK

        def transpose_blk(buf, nv):
            for r in range(nv // 2):
                for g in range(2 * D // _L):
                    vvec = jnp.full((_L,), 2 * r + g // (D // _L), jnp.int32)
                    vals = plsc.load_gather(
                        in_v.at[buf], [dvecs[g % (D // _L)], vvec]
                    )
                    out_v[buf, r, pl.ds(g * _L, _L)] = vals

        def read_blk(t, buf):
            return pltpu.async_copy(
                tab_hbm.at[:, pl.ds(t * _BLK, _BLK)], in_v.at[buf],
                isem.at[buf],
            )

        def write_blk(t, buf):
            return pltpu.async_copy(
                out_v.at[buf],
                scr_hbm.at[pl.ds(t * (_BLK // 2), _BLK // 2)],
                osem.at[buf],
            )

        @pl.loop(wid, TV - 1, step=2 * NW)
        def _blk(t):
            r0 = read_blk(t, 0)

            @pl.when(t + NW < TV - 1)
            def _issue1():
                read_blk(t + NW, 1)

            r0.wait()
            transpose_blk(0, _BLK)
            w0 = write_blk(t, 0)

            @pl.when(t + NW < TV - 1)
            def _second():
                pltpu.make_async_copy(
                    tab_hbm.at[:, pl.ds((t + NW) * _BLK, _BLK)],
                    in_v.at[1], isem.at[1],
                ).wait()
                transpose_blk(1, _BLK)
                write_blk(t + NW, 1).wait()

            w0.wait()

        @pl.when(wid == (TV - 1) % NW)
        def _tail():
            hs = [
                pltpu.async_copy(
                    tab_hbm.at[d, pl.ds((TV - 1) * _BLK, rem)],
                    in_v.at[0, d, pl.ds(0, rem)],
                    isem.at[0],
                )
                for d in range(D)
            ]
            for h in hs:
                h.wait()
            transpose_blk(0, rem)
            pltpu.async_copy(
                out_v.at[0, pl.ds(0, rem // 2)],
                scr_hbm.at[pl.ds((TV - 1) * (_BLK // 2), rem // 2)],
                osem.at[0],
            ).wait()

    return pack_kernel


def _build_gather(S, Bdim, D, VP, NC, NW):
    NBUF = 2
    mesh = plsc.VectorSubcoreMesh(core_axis_name="c", subcore_axis_name="s")

    @functools.partial(
        pl.kernel,
        out_type=jax.ShapeDtypeStruct((S, D, Bdim), jnp.float32),
        mesh=mesh,
        scratch_types=[
            pltpu.VMEM((S, _BLK), jnp.int32),
            pltpu.VMEM((NBUF, _BLK), jnp.int32),
            pltpu.VMEM((NBUF, _BLK, 2 * D), jnp.float32),
            pltpu.VMEM((NBUF, D, _BLK), jnp.float32),
            pltpu.SemaphoreType.DMA((NBUF,)),
            pltpu.SemaphoreType.DMA((NBUF,)),
        ],
        compiler_params=_PARAMS,
    )
    def gather_kernel(scr_hbm, idx_hbm, out_hbm, idx_v, pidx_v, rows_v,
                      tr_v, gsem, osem):
        wid = lax.axis_index("s") * NC + lax.axis_index("c")
        iota = lax.iota(jnp.int32, _L)
        pltpu.sync_copy(idx_hbm.at[:, wid], idx_v)

        @pl.loop(0, S, step=NBUF)
        def _sgroup(s0):
            handles = []
            for b in range(NBUF):
                for g in range(_BLK // _L):
                    sl = pl.ds(g * _L, _L)
                    pidx_v[b, sl] = lax.shift_right_logical(
                        idx_v[s0 + b, sl], 1
                    )
                handles.append(pltpu.async_copy(
                    scr_hbm.at[pidx_v.at[b]], rows_v.at[b], gsem.at[b]
                ))
            writes = []
            for b in range(NBUF):
                handles[b].wait()
                for g in range(_BLK // _L):
                    sl = pl.ds(g * _L, _L)
                    rvec = g * _L + iota
                    par = lax.bitwise_and(idx_v[s0 + b, sl], 1) * D
                    for d in range(D):
                        vals = plsc.load_gather(
                            rows_v.at[b], [rvec, par + d]
                        )
                        tr_v[b, d, sl] = vals
                writes.append(pltpu.async_copy(
                    tr_v.at[b],
                    out_hbm.at[s0 + b, :, pl.ds(wid * _BLK, _BLK)],
                    osem.at[b],
                ))
            for w in writes:
                w.wait()

    return gather_kernel


def kernel(sentence, table):
    B, S = sentence.shape
    V, D = table.shape

    info = plsc.get_sparse_core_info()
    NC, NS = info.num_cores, info.num_subcores
    NW = NC * NS
    assert B % (NW * _BLK) == 0 and S % 2 == 0

    TV = (V + _BLK - 1) // _BLK
    VP = TV * _BLK

    tab_t = table.T
    packed = _build_pack(D, V, NC, NW)(tab_t)

    sidx = sentence.T.reshape(S, B // _BLK, _BLK).astype(jnp.int32)
    out_sdb = _build_gather(S, B, D, VP, NC, NW)(packed, sidx)
    return jnp.transpose(out_sdb, (2, 0, 1))


# speedup vs baseline: 2.8344x; 2.8344x over previous
import functools

import jax
import jax.numpy as jnp
from jax import lax
from jax.experimental import pallas as pl
from jax.experimental.pallas import tpu as pltpu
from jax.experimental.pallas import tpu_sc as plsc

_CHUNK = 128
_NBUF = 8


def _build_lookup(NW, NCHUNK, CHUNK, V, D, NC):
    N = NW * NCHUNK * CHUNK
    mesh = plsc.VectorSubcoreMesh(core_axis_name="c", subcore_axis_name="s")

    @functools.partial(
        pl.kernel,
        out_type=jax.ShapeDtypeStruct((N, D), jnp.float32),
        mesh=mesh,
        scratch_types=[
            pltpu.VMEM((NCHUNK, CHUNK), jnp.int32),
            pltpu.VMEM((_NBUF, CHUNK, D), jnp.float32),
            pltpu.SemaphoreType.DMA((_NBUF,)),
            pltpu.SemaphoreType.DMA((_NBUF,)),
        ],
        compiler_params=pltpu.CompilerParams(use_tc_tiling_on_sc=False),
    )
    def emb_kernel(table_hbm, idx_hbm, out_hbm, idx_v, rows_v, gsem, osem):
        wid = lax.axis_index("s") * NC + lax.axis_index("c")
        base = wid * (NCHUNK * CHUNK)
        pltpu.sync_copy(idx_hbm.at[wid], idx_v)

        @pl.loop(0, NCHUNK, step=_NBUF)
        def _chunk_group(c0):
            gathers = [
                pltpu.async_copy(
                    table_hbm.at[idx_v.at[c0 + b]], rows_v.at[b], gsem.at[b]
                )
                for b in range(_NBUF)
            ]
            writes = []
            for b in range(_NBUF):
                gathers[b].wait()
                writes.append(
                    pltpu.async_copy(
                        rows_v.at[b],
                        out_hbm.at[pl.ds(base + (c0 + b) * CHUNK, CHUNK)],
                        osem.at[b],
                    )
                )
            for w in writes:
                w.wait()

    return emb_kernel


def kernel(sentence, table):
    B, S = sentence.shape
    V, D = table.shape
    N = B * S

    info = plsc.get_sparse_core_info()
    NC, NS = info.num_cores, info.num_subcores
    NW = NC * NS
    assert N % (NW * _CHUNK) == 0
    NCHUNK = N // (NW * _CHUNK)
    assert NCHUNK % _NBUF == 0

    idx = sentence.reshape(NW, NCHUNK, _CHUNK).astype(jnp.int32)
    out = _build_lookup(NW, NCHUNK, _CHUNK, V, D, NC)(table, idx)
    return out.reshape(B, S, D)
